# baseline (speedup 1.0000x reference)
import jax
import jax.numpy as jnp
from jax import lax
from jax.experimental import pallas as pl
from jax.experimental.pallas import tpu as pltpu
from jax.experimental.pallas import tpu_sc as plsc

BATCH = 16384
HIST = 200
DIM = 32
NC, NS = 2, 16
NW = NC * NS
TBT = 4
CT = DIM // 8


def _gather_body(idx_hbm, wl_hbm, o5_hbm,
                 idx_v0, rows0, tb0, si0, sg0, so0):
    wid = lax.axis_index("s") * NC + lax.axis_index("c")
    bt0 = wid * TBT

    iota16 = lax.iota(jnp.int32, 16)
    rvecs = [iota16 + (k * 16) for k in range(8)]

    def transpose_unit():
        def cbody(c, carry):
            csplat = jnp.full((16,), 0, jnp.int32) + c
            for k in range(8):
                vals = plsc.load_gather(rows0, [rvecs[k], csplat])
                tb0[c // 8, c % 8, pl.ds(k * 16, 16)] = vals
            return carry

        lax.fori_loop(0, DIM, cbody, 0)

    def hbody(h, carry):
        pltpu.sync_copy(idx_hbm.at[h, pl.ds(bt0, TBT)], idx_v0)
        for uu in range(TBT):
            pltpu.async_copy(wl_hbm.at[idx_v0.at[uu]], rows0, sg0).wait()
            transpose_unit()
            for ct in range(CT):
                pltpu.sync_copy(tb0.at[ct], o5_hbm.at[h, ct, bt0 + uu])
        return carry

    lax.fori_loop(0, HIST, hbody, 0)


def kernel(input, weight):
    idxT3 = jnp.transpose(input).astype(jnp.int32).reshape(HIST, 128, 128)
    mesh = plsc.VectorSubcoreMesh(core_axis_name="c", subcore_axis_name="s")
    o5 = pl.kernel(
        _gather_body,
        mesh=mesh,
        out_type=jax.ShapeDtypeStruct((HIST, CT, 128, 8, 128), jnp.float32),
        scratch_types=[
            pltpu.VMEM((TBT, 128), jnp.int32),
            pltpu.VMEM((128, DIM), jnp.float32),
            pltpu.VMEM((CT, 8, 128), jnp.float32),
            pltpu.SemaphoreType.DMA,
            pltpu.SemaphoreType.DMA,
            pltpu.SemaphoreType.DMA,
        ],
        compiler_params=pltpu.CompilerParams(
            use_tc_tiling_on_sc=False, needs_layout_passes=False),
    )(idxT3, weight)
    return o5.transpose(2, 4, 0, 1, 3).reshape(BATCH, HIST, DIM)


# speedup vs baseline: 1.3404x; 1.3404x over previous
import jax
import jax.numpy as jnp
from jax import lax
from jax.experimental import pallas as pl
from jax.experimental.pallas import tpu as pltpu
from jax.experimental.pallas import tpu_sc as plsc

BATCH = 16384
HIST = 200
DIM = 32
NC, NS = 2, 16
NW = NC * NS
TBT = 4
CT = DIM // 8
H2 = HIST // 2


def _gather_body(idx_hbm, wl_hbm, o5_hbm,
                 idx_v0, idx_v1, rows0, rows1, tb0, tb1,
                 si0, si1, sg0, sg1, so0, so1):
    wid = lax.axis_index("s") * NC + lax.axis_index("c")
    bt0 = wid * TBT

    idx_v = (idx_v0, idx_v1)
    rows = (rows0, rows1)
    tb = (tb0, tb1)
    si = (si0, si1)
    sg = (sg0, sg1)
    so = (so0, so1)

    iota16 = lax.iota(jnp.int32, 16)
    rvecs = [iota16 + (k * 16) for k in range(8)]

    def idx_slice(h):
        return idx_hbm.at[h, pl.ds(bt0, TBT)]

    def fire_idx(h, islot):
        pltpu.async_copy(idx_slice(h), idx_v[islot], si[islot])

    def wait_idx(h, islot):
        pltpu.make_async_copy(idx_slice(h), idx_v[islot], si[islot]).wait()

    def fire_g(islot, uu, rs):
        pltpu.async_copy(wl_hbm.at[idx_v[islot].at[uu]], rows[rs], sg[rs])

    def wait_g(islot, uu, rs):
        pltpu.make_async_copy(wl_hbm.at[idx_v[islot].at[uu]], rows[rs], sg[rs]).wait()

    def fire_out(h, uu, s):
        for ct in range(CT):
            pltpu.async_copy(tb[s].at[ct], o5_hbm.at[h, ct, bt0 + uu], so[s])

    def wait_out(h, uu, s):
        for ct in range(CT):
            pltpu.make_async_copy(
                tb[s].at[ct], o5_hbm.at[h, ct, bt0 + uu], so[s]).wait()

    def transpose_unit(rs, ts):
        def cbody(c, carry):
            csplat = jnp.full((16,), 0, jnp.int32) + c
            for k in range(8):
                vals = plsc.load_gather(rows[rs], [rvecs[k], csplat])
                tb[ts][c // 8, c % 8, pl.ds(k * 16, 16)] = vals
            return carry

        lax.fori_loop(0, DIM, cbody, 0)

    fire_idx(0, 0)
    wait_idx(0, 0)
    fire_g(0, 0, 0)
    fire_idx(1, 1)

    def h2body(h2, carry):
        for hh in range(2):
            h = h2 * 2 + hh
            for uu in range(TBT):
                s = uu % 2
                wait_g(hh, uu, s)
                if uu < TBT - 1:
                    fire_g(hh, uu + 1, 1 - s)
                elif hh == 0:
                    wait_idx(h + 1, 1)
                    fire_g(1, 0, 1 - s)

                    @pl.when(h2 + 1 < H2)
                    def _():
                        fire_idx(h + 2, 0)
                else:
                    @pl.when(h2 + 1 < H2)
                    def _():
                        wait_idx(h + 1, 0)
                        fire_g(0, 0, 1 - s)
                        fire_idx(h + 2, 1)

                if hh == 0 and uu < 2:
                    @pl.when(h2 > 0)
                    def _():
                        wait_out(h, uu, s)
                else:
                    wait_out(h, uu, s)

                transpose_unit(s, s)
                fire_out(h, uu, s)
        return carry

    lax.fori_loop(0, H2, h2body, 0)

    wait_out(HIST - 1, TBT - 2, 0)
    wait_out(HIST - 1, TBT - 1, 1)


def kernel(input, weight):
    idxT3 = jnp.transpose(input).astype(jnp.int32).reshape(HIST, 128, 128)
    mesh = plsc.VectorSubcoreMesh(core_axis_name="c", subcore_axis_name="s")
    o5 = pl.kernel(
        _gather_body,
        mesh=mesh,
        out_type=jax.ShapeDtypeStruct((HIST, CT, 128, 8, 128), jnp.float32),
        scratch_types=[
            pltpu.VMEM((TBT, 128), jnp.int32),
            pltpu.VMEM((TBT, 128), jnp.int32),
            pltpu.VMEM((128, DIM), jnp.float32),
            pltpu.VMEM((128, DIM), jnp.float32),
            pltpu.VMEM((CT, 8, 128), jnp.float32),
            pltpu.VMEM((CT, 8, 128), jnp.float32),
            pltpu.SemaphoreType.DMA,
            pltpu.SemaphoreType.DMA,
            pltpu.SemaphoreType.DMA,
            pltpu.SemaphoreType.DMA,
            pltpu.SemaphoreType.DMA,
            pltpu.SemaphoreType.DMA,
        ],
        compiler_params=pltpu.CompilerParams(
            use_tc_tiling_on_sc=False, needs_layout_passes=False),
    )(idxT3, weight)
    return o5.transpose(2, 4, 0, 1, 3).reshape(BATCH, HIST, DIM)


# speedup vs baseline: 1.5898x; 1.1860x over previous
import jax
import jax.numpy as jnp
from jax import lax
from jax.experimental import pallas as pl
from jax.experimental.pallas import tpu as pltpu
from jax.experimental.pallas import tpu_sc as plsc

BATCH = 16384
HIST = 200
DIM = 32
NC, NS = 2, 16
NW = NC * NS
TBT = 4
CT = DIM // 8
H2 = HIST // 2


def _gather_body(idx_hbm, wl_hbm, o5_hbm,
                 idx_v0, idx_v1, rows0, rows1, tb0, tb1,
                 si0, si1, sg0, sg1, so0, so1):
    wid = lax.axis_index("s") * NC + lax.axis_index("c")
    bt0 = wid * TBT

    idx_v = (idx_v0, idx_v1)
    rows = (rows0, rows1)
    tb = (tb0, tb1)
    si = (si0, si1)
    sg = (sg0, sg1)
    so = (so0, so1)

    iota16 = lax.iota(jnp.int32, 16)
    rvecs = [iota16 + (k * 16) for k in range(8)]

    def idx_slice(h):
        return idx_hbm.at[h, pl.ds(bt0, TBT)]

    def fire_idx(h, islot):
        pltpu.async_copy(idx_slice(h), idx_v[islot], si[islot])

    def wait_idx(h, islot):
        pltpu.make_async_copy(idx_slice(h), idx_v[islot], si[islot]).wait()

    def fire_g(islot, uu, rs):
        pltpu.async_copy(wl_hbm.at[idx_v[islot].at[uu]], rows[rs], sg[rs])

    def wait_g(islot, uu, rs):
        pltpu.make_async_copy(wl_hbm.at[idx_v[islot].at[uu]], rows[rs], sg[rs]).wait()

    def fire_out(h, uu, s):
        for ct in range(CT):
            pltpu.async_copy(tb[s].at[ct], o5_hbm.at[h, ct, bt0 + uu], so[s])

    def wait_out(h, uu, s):
        for ct in range(CT):
            pltpu.make_async_copy(
                tb[s].at[ct], o5_hbm.at[h, ct, bt0 + uu], so[s]).wait()

    def transpose_unit(rs, ts):
        def cbody(c2, carry):
            for j in range(2):
                c = c2 * 2 + j
                csplat = jnp.full((16,), 0, jnp.int32) + c
                vals = [plsc.load_gather(rows[rs], [rvecs[k], csplat])
                        for k in range(8)]
                for k in range(8):
                    tb[ts][c // 8, c % 8, pl.ds(k * 16, 16)] = vals[k]
            return carry

        lax.fori_loop(0, DIM // 2, cbody, 0)

    fire_idx(0, 0)
    wait_idx(0, 0)
    fire_g(0, 0, 0)
    fire_idx(1, 1)

    def h2body(h2, carry):
        for hh in range(2):
            h = h2 * 2 + hh
            for uu in range(TBT):
                s = uu % 2
                wait_g(hh, uu, s)
                if uu < TBT - 1:
                    fire_g(hh, uu + 1, 1 - s)
                elif hh == 0:
                    wait_idx(h + 1, 1)
                    fire_g(1, 0, 1 - s)

                    @pl.when(h2 + 1 < H2)
                    def _():
                        fire_idx(h + 2, 0)
                else:
                    @pl.when(h2 + 1 < H2)
                    def _():
                        wait_idx(h + 1, 0)
                        fire_g(0, 0, 1 - s)
                        fire_idx(h + 2, 1)

                if hh == 0 and uu < 2:
                    @pl.when(h2 > 0)
                    def _():
                        wait_out(h, uu, s)
                else:
                    wait_out(h, uu, s)

                transpose_unit(s, s)
                fire_out(h, uu, s)
        return carry

    lax.fori_loop(0, H2, h2body, 0)

    wait_out(HIST - 1, TBT - 2, 0)
    wait_out(HIST - 1, TBT - 1, 1)


def kernel(input, weight):
    idxT3 = jnp.transpose(input).astype(jnp.int32).reshape(HIST, 128, 128)
    mesh = plsc.VectorSubcoreMesh(core_axis_name="c", subcore_axis_name="s")
    o5 = pl.kernel(
        _gather_body,
        mesh=mesh,
        out_type=jax.ShapeDtypeStruct((HIST, CT, 128, 8, 128), jnp.float32),
        scratch_types=[
            pltpu.VMEM((TBT, 128), jnp.int32),
            pltpu.VMEM((TBT, 128), jnp.int32),
            pltpu.VMEM((128, DIM), jnp.float32),
            pltpu.VMEM((128, DIM), jnp.float32),
            pltpu.VMEM((CT, 8, 128), jnp.float32),
            pltpu.VMEM((CT, 8, 128), jnp.float32),
            pltpu.SemaphoreType.DMA,
            pltpu.SemaphoreType.DMA,
            pltpu.SemaphoreType.DMA,
            pltpu.SemaphoreType.DMA,
            pltpu.SemaphoreType.DMA,
            pltpu.SemaphoreType.DMA,
        ],
        compiler_params=pltpu.CompilerParams(
            use_tc_tiling_on_sc=False, needs_layout_passes=False),
    )(idxT3, weight)
    return o5.transpose(2, 4, 0, 1, 3).reshape(BATCH, HIST, DIM)
